# baseline (speedup 1.0000x reference)
import jax
import jax.numpy as jnp
from jax import lax
from jax.experimental import pallas as pl
from jax.experimental.pallas import tpu as pltpu
from jax.experimental.pallas import tpu_sc as plsc

_N = 10000
_E = 320000
_D = 128
_EPS = 0.0
_BN_EPS = 1e-5

_NC = 2
_NS = 16
_NW = _NC * _NS

_CH = 128
_T = 80
_NBLK = 4
_TB = _T // _NBLK
_NB = 2
_EPW = _T * _CH
_EW = _E // _NW
_NACC = 10112
_ZR = _NACC // _NS


def _sc_agg_body(h, srcp, dstp, zeros, out, sidx, didx, rows, acc,
                 sem_g0, sem_g1, sem_s):
    c = lax.axis_index("c")
    s = lax.axis_index("s")
    w = c * _NS + s

    pltpu.sync_copy(zeros, acc.at[pl.ds(s * _ZR, _ZR)])
    plsc.subcore_barrier()

    def start_g(t, sem):
        pltpu.async_copy(h.at[sidx.at[t]], rows.at[lax.rem(t, _NB)], sem)

    def start_s(t):
        pltpu.async_copy(rows.at[lax.rem(t, _NB)], acc.at[didx.at[t]],
                         sem_s, add=True)

    def wait(sem):
        pltpu.make_async_copy(zeros.at[pl.ds(0, _CH)], rows.at[0], sem).wait()

    def halfstep(t, sem, last):
        wait(sem)
        start_s(t)
        wait(sem_s)
        if not last:
            start_g(t + 2, sem)

    def step(u, carry):
        halfstep(2 * u, sem_g0, False)
        halfstep(2 * u + 1, sem_g1, False)
        return carry

    for b in range(_NBLK):
        pltpu.sync_copy(srcp.at[w, b], sidx)
        pltpu.sync_copy(dstp.at[w, b], didx)
        start_g(0, sem_g0)
        start_g(1, sem_g1)
        lax.fori_loop(0, _TB // 2 - 1, step, 0)
        halfstep(_TB - 2, sem_g0, True)
        halfstep(_TB - 1, sem_g1, True)
    plsc.subcore_barrier()

    pltpu.sync_copy(acc.at[pl.ds(s * _ZR, _ZR)], out.at[c, pl.ds(s * _ZR, _ZR)])


def _sc_agg(h, srcp, dstp, zeros):
    mesh = plsc.VectorSubcoreMesh(core_axis_name="c", subcore_axis_name="s")
    return pl.kernel(
        _sc_agg_body,
        out_type=jax.ShapeDtypeStruct((_NC, _NACC, _D), jnp.float32),
        mesh=mesh,
        scratch_types=[
            pltpu.VMEM((_TB, _CH), jnp.int32),
            pltpu.VMEM((_TB, _CH), jnp.int32),
            pltpu.VMEM((_NB, _CH, _D), jnp.float32),
            pltpu.VMEM_SHARED((_NACC, _D), jnp.float32),
            pltpu.SemaphoreType.DMA,
            pltpu.SemaphoreType.DMA,
            pltpu.SemaphoreType.DMA,
        ],
    )(h, srcp, dstp, zeros)


def _mlp_bn_body(h, a, w1, b1, w2, b2, gamma, beta, o):
    z = h[...] * (1.0 + _EPS) + a[0, : _N] + a[1, : _N]
    z = jnp.maximum(jnp.dot(z, w1[...], preferred_element_type=jnp.float32) + b1[...], 0.0)
    z = jnp.dot(z, w2[...], preferred_element_type=jnp.float32) + b2[...]
    mu = jnp.mean(z, axis=0, keepdims=True)
    var = jnp.mean(jnp.square(z - mu), axis=0, keepdims=True)
    zn = gamma[...] * (z - mu) * lax.rsqrt(var + _BN_EPS) + beta[...]
    o[...] = jnp.maximum(zn, 0.0)


def _mlp_last_body(h, a, w1, b1, w2, b2, o):
    z = h[...] * (1.0 + _EPS) + a[0, : _N] + a[1, : _N]
    z = jnp.maximum(jnp.dot(z, w1[...], preferred_element_type=jnp.float32) + b1[...], 0.0)
    o[...] = jnp.dot(z, w2[...], preferred_element_type=jnp.float32) + b2[...]


def _mlp_bn(h, a, w1, b1, w2, b2, gamma, beta):
    return pl.pallas_call(
        _mlp_bn_body,
        out_shape=jax.ShapeDtypeStruct((_N, _D), jnp.float32),
    )(h, a, w1, b1.reshape(1, -1), w2, b2.reshape(1, -1),
      gamma.reshape(1, -1), beta.reshape(1, -1))


def _mlp_last(h, a, w1, b1, w2, b2):
    return pl.pallas_call(
        _mlp_last_body,
        out_shape=jax.ShapeDtypeStruct((_N, _D), jnp.float32),
    )(h, a, w1, b1.reshape(1, -1), w2, b2.reshape(1, -1))


def kernel(x, edge_index, params):
    src = edge_index[0]
    dst = edge_index[1]
    srcp = jnp.pad(src.reshape(_NW, _EW), ((0, 0), (0, _EPW - _EW)))
    srcp = srcp.reshape(_NW, _NBLK, _TB, _CH)
    dstp = jnp.pad(dst.reshape(_NW, _EW), ((0, 0), (0, _EPW - _EW)),
                   constant_values=_N)
    dstp = dstp.reshape(_NW, _NBLK, _TB, _CH)
    zeros = jnp.zeros((_ZR, _D), jnp.float32)

    h = x
    num_layers = len(params["convs"])
    for i in range(num_layers):
        w1, b1, w2, b2 = params["convs"][i]
        a = _sc_agg(h, srcp, dstp, zeros)
        if i < num_layers - 1:
            gamma, beta = params["bns"][i]
            h = _mlp_bn(h, a, w1, b1, w2, b2, gamma, beta)
        else:
            h = _mlp_last(h, a, w1, b1, w2, b2)
    return h


# speedup vs baseline: 1.0177x; 1.0177x over previous
import jax
import jax.numpy as jnp
from jax import lax
from jax.experimental import pallas as pl
from jax.experimental.pallas import tpu as pltpu
from jax.experimental.pallas import tpu_sc as plsc

_N = 10000
_E = 320000
_D = 128
_EPS = 0.0
_BN_EPS = 1e-5

_NC = 2
_NS = 16
_NW = _NC * _NS

_CH = 128
_T = 80
_NBLK = 8
_TB = _T // _NBLK
_NB = 2
_EPW = _T * _CH
_EW = _E // _NW
_NACC = 10112
_ZR = _NACC // _NS


def _sc_agg_body(h, srcp, dstp, zeros, out, sidx, didx, rows, acc,
                 sem_g, sem_s, sem_i):
    c = lax.axis_index("c")
    s = lax.axis_index("s")
    w = c * _NS + s

    pltpu.sync_copy(zeros, acc.at[pl.ds(s * _ZR, _ZR)])
    plsc.subcore_barrier()

    def idx_at(buf, t):
        return buf.at[lax.rem(t // _TB, 2), lax.rem(t, _TB)]

    def start_g(t):
        pltpu.async_copy(h.at[idx_at(sidx, t)], rows.at[lax.rem(t, _NB)],
                         sem_g)

    def start_s(t):
        pltpu.async_copy(rows.at[lax.rem(t, _NB)], acc.at[idx_at(didx, t)],
                         sem_s, add=True)

    def wait(sem):
        pltpu.make_async_copy(zeros.at[pl.ds(0, _CH)], rows.at[0], sem).wait()

    def wait_idx():
        pltpu.make_async_copy(srcp.at[w, 0], sidx.at[0], sem_i).wait()
        pltpu.make_async_copy(dstp.at[w, 0], didx.at[0], sem_i).wait()

    def step(t, carry):
        wait(sem_g)
        start_s(t)
        wait(sem_s)
        start_g(t + 2)
        return carry

    pltpu.sync_copy(srcp.at[w, 0], sidx.at[0])
    pltpu.sync_copy(dstp.at[w, 0], didx.at[0])
    start_g(0)
    start_g(1)
    for b in range(_NBLK):
        not_last = b + 1 < _NBLK
        if not_last:
            pltpu.async_copy(srcp.at[w, b + 1], sidx.at[(b + 1) % 2], sem_i)
            pltpu.async_copy(dstp.at[w, b + 1], didx.at[(b + 1) % 2], sem_i)
        lax.fori_loop(b * _TB, b * _TB + _TB - 2, step, 0)
        if not_last:
            wait_idx()
            step(b * _TB + _TB - 2, 0)
            step(b * _TB + _TB - 1, 0)
        else:
            for t in (b * _TB + _TB - 2, b * _TB + _TB - 1):
                wait(sem_g)
                start_s(t)
                wait(sem_s)
    plsc.subcore_barrier()

    pltpu.sync_copy(acc.at[pl.ds(s * _ZR, _ZR)], out.at[c, pl.ds(s * _ZR, _ZR)])


def _sc_agg(h, srcp, dstp, zeros):
    mesh = plsc.VectorSubcoreMesh(core_axis_name="c", subcore_axis_name="s")
    return pl.kernel(
        _sc_agg_body,
        out_type=jax.ShapeDtypeStruct((_NC, _NACC, _D), jnp.float32),
        mesh=mesh,
        scratch_types=[
            pltpu.VMEM((2, _TB, _CH), jnp.int32),
            pltpu.VMEM((2, _TB, _CH), jnp.int32),
            pltpu.VMEM((_NB, _CH, _D), jnp.float32),
            pltpu.VMEM_SHARED((_NACC, _D), jnp.float32),
            pltpu.SemaphoreType.DMA,
            pltpu.SemaphoreType.DMA,
            pltpu.SemaphoreType.DMA,
        ],
    )(h, srcp, dstp, zeros)


def _mlp_bn_body(h, a, w1, b1, w2, b2, gamma, beta, o):
    z = h[...] * (1.0 + _EPS) + a[0, : _N] + a[1, : _N]
    z = jnp.maximum(jnp.dot(z, w1[...], preferred_element_type=jnp.float32) + b1[...], 0.0)
    z = jnp.dot(z, w2[...], preferred_element_type=jnp.float32) + b2[...]
    mu = jnp.mean(z, axis=0, keepdims=True)
    var = jnp.mean(jnp.square(z - mu), axis=0, keepdims=True)
    zn = gamma[...] * (z - mu) * lax.rsqrt(var + _BN_EPS) + beta[...]
    o[...] = jnp.maximum(zn, 0.0)


def _mlp_last_body(h, a, w1, b1, w2, b2, o):
    z = h[...] * (1.0 + _EPS) + a[0, : _N] + a[1, : _N]
    z = jnp.maximum(jnp.dot(z, w1[...], preferred_element_type=jnp.float32) + b1[...], 0.0)
    o[...] = jnp.dot(z, w2[...], preferred_element_type=jnp.float32) + b2[...]


def _mlp_bn(h, a, w1, b1, w2, b2, gamma, beta):
    return pl.pallas_call(
        _mlp_bn_body,
        out_shape=jax.ShapeDtypeStruct((_N, _D), jnp.float32),
    )(h, a, w1, b1.reshape(1, -1), w2, b2.reshape(1, -1),
      gamma.reshape(1, -1), beta.reshape(1, -1))


def _mlp_last(h, a, w1, b1, w2, b2):
    return pl.pallas_call(
        _mlp_last_body,
        out_shape=jax.ShapeDtypeStruct((_N, _D), jnp.float32),
    )(h, a, w1, b1.reshape(1, -1), w2, b2.reshape(1, -1))


def kernel(x, edge_index, params):
    src = edge_index[0]
    dst = edge_index[1]
    srcp = jnp.pad(src.reshape(_NW, _EW), ((0, 0), (0, _EPW - _EW)))
    srcp = srcp.reshape(_NW, _NBLK, _TB, _CH)
    dstp = jnp.pad(dst.reshape(_NW, _EW), ((0, 0), (0, _EPW - _EW)),
                   constant_values=_N)
    dstp = dstp.reshape(_NW, _NBLK, _TB, _CH)
    zeros = jnp.zeros((_ZR, _D), jnp.float32)

    h = x
    num_layers = len(params["convs"])
    for i in range(num_layers):
        w1, b1, w2, b2 = params["convs"][i]
        a = _sc_agg(h, srcp, dstp, zeros)
        if i < num_layers - 1:
            gamma, beta = params["bns"][i]
            h = _mlp_bn(h, a, w1, b1, w2, b2, gamma, beta)
        else:
            h = _mlp_last(h, a, w1, b1, w2, b2)
    return h


# speedup vs baseline: 1.0221x; 1.0043x over previous
import jax
import jax.numpy as jnp
from jax import lax
from jax.experimental import pallas as pl
from jax.experimental.pallas import tpu as pltpu
from jax.experimental.pallas import tpu_sc as plsc

_N = 10000
_E = 320000
_D = 128
_EPS = 0.0
_BN_EPS = 1e-5

_NC = 2
_NS = 16
_NW = _NC * _NS

_CH = 128
_T = 80
_NBLK = 8
_TB = _T // _NBLK
_NB = 2
_EPW = _T * _CH
_EW = _E // _NW
_NACC = 10112
_ZR = _NACC // _NS


def _sc_agg_body(h, srcp, dstp, zeros, out, sidx, didx, rows, acc,
                 sem_g, sem_s, sem_i):
    c = lax.axis_index("c")
    s = lax.axis_index("s")
    w = c * _NS + s

    def idx_at(buf, t):
        return buf.at[lax.rem(t // _TB, 2), lax.rem(t, _TB)]

    def start_g(t):
        pltpu.async_copy(h.at[idx_at(sidx, t)], rows.at[lax.rem(t, _NB)],
                         sem_g)

    def start_s(t):
        pltpu.async_copy(rows.at[lax.rem(t, _NB)], acc.at[idx_at(didx, t)],
                         sem_s, add=True)

    def wait(sem):
        pltpu.make_async_copy(zeros.at[pl.ds(0, _CH)], rows.at[0], sem).wait()

    def wait_idx():
        pltpu.make_async_copy(srcp.at[w, 0], sidx.at[0], sem_i).wait()
        pltpu.make_async_copy(dstp.at[w, 0], didx.at[0], sem_i).wait()

    def step(t, carry):
        wait(sem_g)
        start_s(t)
        wait(sem_s)
        start_g(t + 2)
        return carry

    pltpu.sync_copy(srcp.at[w, 0], sidx.at[0])
    pltpu.sync_copy(dstp.at[w, 0], didx.at[0])
    start_g(0)
    start_g(1)
    pltpu.sync_copy(zeros, acc.at[pl.ds(s * _ZR, _ZR)])
    plsc.subcore_barrier()
    for b in range(_NBLK):
        not_last = b + 1 < _NBLK
        if not_last:
            pltpu.async_copy(srcp.at[w, b + 1], sidx.at[(b + 1) % 2], sem_i)
            pltpu.async_copy(dstp.at[w, b + 1], didx.at[(b + 1) % 2], sem_i)
        lax.fori_loop(b * _TB, b * _TB + _TB - 2, step, 0)
        if not_last:
            wait_idx()
            step(b * _TB + _TB - 2, 0)
            step(b * _TB + _TB - 1, 0)
        else:
            for t in (b * _TB + _TB - 2, b * _TB + _TB - 1):
                wait(sem_g)
                start_s(t)
                wait(sem_s)
    plsc.subcore_barrier()

    pltpu.sync_copy(acc.at[pl.ds(s * _ZR, _ZR)], out.at[c, pl.ds(s * _ZR, _ZR)])


def _sc_agg(h, srcp, dstp, zeros):
    mesh = plsc.VectorSubcoreMesh(core_axis_name="c", subcore_axis_name="s")
    return pl.kernel(
        _sc_agg_body,
        out_type=jax.ShapeDtypeStruct((_NC, _NACC, _D), jnp.float32),
        mesh=mesh,
        scratch_types=[
            pltpu.VMEM((2, _TB, _CH), jnp.int32),
            pltpu.VMEM((2, _TB, _CH), jnp.int32),
            pltpu.VMEM((_NB, _CH, _D), jnp.float32),
            pltpu.VMEM_SHARED((_NACC, _D), jnp.float32),
            pltpu.SemaphoreType.DMA,
            pltpu.SemaphoreType.DMA,
            pltpu.SemaphoreType.DMA,
        ],
    )(h, srcp, dstp, zeros)


def _mlp_bn_body(h, a, w1, b1, w2, b2, gamma, beta, o):
    z = h[...] * (1.0 + _EPS) + a[0, : _N] + a[1, : _N]
    z = jnp.maximum(jnp.dot(z, w1[...], preferred_element_type=jnp.float32) + b1[...], 0.0)
    z = jnp.dot(z, w2[...], preferred_element_type=jnp.float32) + b2[...]
    mu = jnp.mean(z, axis=0, keepdims=True)
    var = jnp.mean(jnp.square(z - mu), axis=0, keepdims=True)
    zn = gamma[...] * (z - mu) * lax.rsqrt(var + _BN_EPS) + beta[...]
    o[...] = jnp.maximum(zn, 0.0)


def _mlp_last_body(h, a, w1, b1, w2, b2, o):
    z = h[...] * (1.0 + _EPS) + a[0, : _N] + a[1, : _N]
    z = jnp.maximum(jnp.dot(z, w1[...], preferred_element_type=jnp.float32) + b1[...], 0.0)
    o[...] = jnp.dot(z, w2[...], preferred_element_type=jnp.float32) + b2[...]


def _mlp_bn(h, a, w1, b1, w2, b2, gamma, beta):
    return pl.pallas_call(
        _mlp_bn_body,
        out_shape=jax.ShapeDtypeStruct((_N, _D), jnp.float32),
    )(h, a, w1, b1.reshape(1, -1), w2, b2.reshape(1, -1),
      gamma.reshape(1, -1), beta.reshape(1, -1))


def _mlp_last(h, a, w1, b1, w2, b2):
    return pl.pallas_call(
        _mlp_last_body,
        out_shape=jax.ShapeDtypeStruct((_N, _D), jnp.float32),
    )(h, a, w1, b1.reshape(1, -1), w2, b2.reshape(1, -1))


def kernel(x, edge_index, params):
    src = edge_index[0]
    dst = edge_index[1]
    srcp = jnp.pad(src.reshape(_NW, _EW), ((0, 0), (0, _EPW - _EW)))
    srcp = srcp.reshape(_NW, _NBLK, _TB, _CH)
    dstp = jnp.pad(dst.reshape(_NW, _EW), ((0, 0), (0, _EPW - _EW)),
                   constant_values=_N)
    dstp = dstp.reshape(_NW, _NBLK, _TB, _CH)
    zeros = jnp.zeros((_ZR, _D), jnp.float32)

    h = x
    num_layers = len(params["convs"])
    for i in range(num_layers):
        w1, b1, w2, b2 = params["convs"][i]
        a = _sc_agg(h, srcp, dstp, zeros)
        if i < num_layers - 1:
            gamma, beta = params["bns"][i]
            h = _mlp_bn(h, a, w1, b1, w2, b2, gamma, beta)
        else:
            h = _mlp_last(h, a, w1, b1, w2, b2)
    return h
